# Initial kernel scaffold; baseline (speedup 1.0000x reference)
import jax
import jax.numpy as jnp
from jax.experimental import pallas as pl


def kernel(all_ts, W, cumhisto):
    raise NotImplementedError("write your pallas kernel here")


# speedup vs baseline: 17.7163x; 17.7163x over previous
import jax
import jax.numpy as jnp
from jax.experimental import pallas as pl
from jax.experimental.pallas import tpu as pltpu

_N = 1024
_D = 256
_T = 4096
_LAM = 0.25


def _body(ts_ref, W_ref, h_ref, out_ref, tsn_s, W_s, h_s, nrm_s):
    W_s[...] = W_ref[...]
    h_s[...] = h_ref[...]
    ts = ts_ref[...]
    tsn_s[...] = ts / jnp.sqrt(jnp.sum(ts * ts, axis=1, keepdims=True))
    Wv = W_s[...]
    nrm_s[...] = jnp.sqrt(jnp.sum(Wv * Wv, axis=1)).reshape(1, _N)

    lane = jax.lax.broadcasted_iota(jnp.int32, (1, _N), 1)

    def step(t, carry):
        s, _ = carry
        tsn = tsn_s[pl.ds(t, 1), :]
        beta = jax.lax.dot_general(
            tsn, W_s[...], (((1,), (1,)), ((), ())),
            preferred_element_type=jnp.float32)
        beta = beta / nrm_s[...]
        h = h_s[...]
        gain = jnp.exp(_LAM * (1.0 - 1024.0 * h / s))
        score = gain * beta
        n_star = jnp.argmax(score).astype(jnp.int32)
        onehot = lane == n_star
        bstar = jnp.sum(jnp.where(onehot, beta, 0.0))
        hstar = jnp.sum(jnp.where(onehot, h, 0.0))
        alpha = 0.01 / (1.0 + hstar / 20000.0)
        Ck = W_s[pl.ds(n_star, 1), :]
        new_row = Ck + alpha * bstar * (tsn - Ck)
        W_s[pl.ds(n_star, 1), :] = new_row
        nn = jnp.sqrt(jnp.sum(new_row * new_row))
        nrm_s[...] = jnp.where(onehot, nn, nrm_s[...])
        h_s[...] = jnp.where(onehot, h + 1.0, h)
        return (s + 1.0, n_star)

    s0 = jnp.sum(h_ref[...])
    _, n_last = jax.lax.fori_loop(0, _T, step, (s0, jnp.int32(0)))
    out_ref[0, 0] = n_last


def kernel(all_ts, W, cumhisto):
    h2 = cumhisto.reshape(1, _N)
    out = pl.pallas_call(
        _body,
        out_shape=jax.ShapeDtypeStruct((1, 1), jnp.int32),
        in_specs=[
            pl.BlockSpec(memory_space=pltpu.VMEM),
            pl.BlockSpec(memory_space=pltpu.VMEM),
            pl.BlockSpec(memory_space=pltpu.VMEM),
        ],
        out_specs=pl.BlockSpec(memory_space=pltpu.SMEM),
        scratch_shapes=[
            pltpu.VMEM((_T, _D), jnp.float32),
            pltpu.VMEM((_N, _D), jnp.float32),
            pltpu.VMEM((1, _N), jnp.float32),
            pltpu.VMEM((1, _N), jnp.float32),
        ],
    )(all_ts, W, h2)
    return out[0, 0]


# speedup vs baseline: 19.9903x; 1.1284x over previous
import jax
import jax.numpy as jnp
from jax.experimental import pallas as pl
from jax.experimental.pallas import tpu as pltpu

_N = 1024
_D = 256
_T = 4096
_LAM = 0.25


def _body(ts_ref, W_ref, h_ref, out_ref, tsn_s, W_s, h_s, nrm_s):
    W_s[...] = W_ref[...]
    h_s[...] = h_ref[...]
    ts = ts_ref[...]
    tsn_s[...] = ts / jnp.sqrt(jnp.sum(ts * ts, axis=1, keepdims=True))
    Wv = W_s[...]
    nrm_s[...] = jnp.sqrt(jnp.sum(Wv * Wv, axis=1)).reshape(8, 128)

    ridx = jax.lax.broadcasted_iota(jnp.int32, (8, 128), 0)
    cidx = jax.lax.broadcasted_iota(jnp.int32, (8, 128), 1)
    flat = ridx * 128 + cidx

    def step(t, carry):
        s, _ = carry
        tsn = tsn_s[pl.ds(t, 1), :]
        dot = jax.lax.dot_general(
            tsn, W_s[...], (((1,), (1,)), ((), ())),
            preferred_element_type=jnp.float32)
        beta = dot.reshape(8, 128) / nrm_s[...]
        h = h_s[...]
        gain = jnp.exp(_LAM * (1.0 - 1024.0 * h / s))
        score = gain * beta
        n_star = jnp.argmax(score).astype(jnp.int32)
        onehot = flat == n_star
        bstar = jnp.sum(jnp.where(onehot, beta, 0.0))
        hstar = jnp.sum(jnp.where(onehot, h, 0.0))
        alpha = 0.01 / (1.0 + hstar / 20000.0)
        Ck = W_s[pl.ds(n_star, 1), :]
        new_row = Ck + alpha * bstar * (tsn - Ck)
        W_s[pl.ds(n_star, 1), :] = new_row
        nn = jnp.sqrt(jnp.sum(new_row * new_row))
        nrm_s[...] = jnp.where(onehot, nn, nrm_s[...])
        h_s[...] = jnp.where(onehot, h + 1.0, h)
        return (s + 1.0, n_star)

    s0 = jnp.sum(h_ref[...])
    _, n_last = jax.lax.fori_loop(0, _T, step, (s0, jnp.int32(0)), unroll=2)
    out_ref[0, 0] = n_last


def kernel(all_ts, W, cumhisto):
    h2 = cumhisto.reshape(8, 128)
    out = pl.pallas_call(
        _body,
        out_shape=jax.ShapeDtypeStruct((1, 1), jnp.int32),
        in_specs=[
            pl.BlockSpec(memory_space=pltpu.VMEM),
            pl.BlockSpec(memory_space=pltpu.VMEM),
            pl.BlockSpec(memory_space=pltpu.VMEM),
        ],
        out_specs=pl.BlockSpec(memory_space=pltpu.SMEM),
        scratch_shapes=[
            pltpu.VMEM((_T, _D), jnp.float32),
            pltpu.VMEM((_N, _D), jnp.float32),
            pltpu.VMEM((8, 128), jnp.float32),
            pltpu.VMEM((8, 128), jnp.float32),
        ],
    )(all_ts, W, h2)
    return out[0, 0]


# speedup vs baseline: 20.8441x; 1.0427x over previous
import jax
import jax.numpy as jnp
from jax.experimental import pallas as pl
from jax.experimental.pallas import tpu as pltpu

_N = 1024
_D = 256
_T = 4096
_LAM = 0.25


def _dot(lhs, rhs):
    return jax.lax.dot_general(
        lhs, rhs, (((1,), (1,)), ((), ())),
        preferred_element_type=jnp.float32)


def _body(ts_ref, W_ref, h_ref, out_ref, tsn_s, W_s, h_s, nrm_s, buf_s):
    W_s[...] = W_ref[...]
    h_s[...] = h_ref[...]
    ts = ts_ref[...]
    tsn_s[...] = ts / jnp.sqrt(jnp.sum(ts * ts, axis=1, keepdims=True))
    Wv = W_s[...]
    nrm_s[...] = jnp.sqrt(jnp.sum(Wv * Wv, axis=1)).reshape(8, 128)
    buf_s[...] = jnp.zeros((8, _D), jnp.float32)

    ridx = jax.lax.broadcasted_iota(jnp.int32, (8, 128), 0)
    cidx = jax.lax.broadcasted_iota(jnp.int32, (8, 128), 1)
    flat = ridx * 128 + cidx

    dot0 = _dot(tsn_s[pl.ds(0, 1), :], W_s[...]).reshape(8, 128)

    def step(t, carry):
        s, r_prev, dot_stale = carry
        tsn = tsn_s[pl.ds(t, 1), :]
        corr = _dot(tsn, buf_s[...])
        c = jnp.max(corr)
        oh_prev = flat == r_prev
        dotv = jnp.where(oh_prev, c, dot_stale)
        tnext = jnp.minimum(t + 1, _T - 1)
        dot_next = _dot(tsn_s[pl.ds(tnext, 1), :], W_s[...]).reshape(8, 128)
        beta = dotv / nrm_s[...]
        h = h_s[...]
        gain = jnp.exp(_LAM * (1.0 - 1024.0 * h / s))
        score = gain * beta
        n_star = jnp.argmax(score).astype(jnp.int32)
        onehot = flat == n_star
        bstar = jnp.sum(jnp.where(onehot, beta, 0.0))
        hstar = jnp.sum(jnp.where(onehot, h, 0.0))
        alpha = 0.01 / (1.0 + hstar / 20000.0)
        Ck = W_s[pl.ds(n_star, 1), :]
        new_row = Ck + alpha * bstar * (tsn - Ck)
        W_s[pl.ds(n_star, 1), :] = new_row
        buf_s[...] = jnp.broadcast_to(new_row, (8, _D))
        nn = jnp.sqrt(jnp.sum(new_row * new_row))
        nrm_s[...] = jnp.where(onehot, nn, nrm_s[...])
        h_s[...] = jnp.where(onehot, h + 1.0, h)
        return (s + 1.0, n_star, dot_next)

    s0 = jnp.sum(h_ref[...])
    carry = (s0, jnp.int32(-1), dot0)
    _, n_last, _ = jax.lax.fori_loop(0, _T, step, carry, unroll=2)
    out_ref[0, 0] = n_last


def kernel(all_ts, W, cumhisto):
    h2 = cumhisto.reshape(8, 128)
    out = pl.pallas_call(
        _body,
        out_shape=jax.ShapeDtypeStruct((1, 1), jnp.int32),
        in_specs=[
            pl.BlockSpec(memory_space=pltpu.VMEM),
            pl.BlockSpec(memory_space=pltpu.VMEM),
            pl.BlockSpec(memory_space=pltpu.VMEM),
        ],
        out_specs=pl.BlockSpec(memory_space=pltpu.SMEM),
        scratch_shapes=[
            pltpu.VMEM((_T, _D), jnp.float32),
            pltpu.VMEM((_N, _D), jnp.float32),
            pltpu.VMEM((8, 128), jnp.float32),
            pltpu.VMEM((8, 128), jnp.float32),
            pltpu.VMEM((8, _D), jnp.float32),
        ],
    )(all_ts, W, h2)
    return out[0, 0]


# speedup vs baseline: 22.6556x; 1.0869x over previous
import jax
import jax.numpy as jnp
from jax.experimental import pallas as pl
from jax.experimental.pallas import tpu as pltpu

_N = 1024
_D = 256
_T = 4096
_LAM = 0.25


def _dot(lhs, rhs):
    return jax.lax.dot_general(
        lhs, rhs, (((1,), (1,)), ((), ())),
        preferred_element_type=jnp.float32)


def _body(ts_ref, W_ref, h_ref, out_ref, tsn_s, W_s, h_s, nrm_s):
    W_s[...] = W_ref[...]
    h_s[...] = h_ref[...]
    ts = ts_ref[...]
    tsn_s[...] = ts / jnp.sqrt(jnp.sum(ts * ts, axis=1, keepdims=True))
    Wv = W_s[...]
    nrm_s[...] = jnp.sqrt(jnp.sum(Wv * Wv, axis=1)).reshape(8, 128)

    ridx = jax.lax.broadcasted_iota(jnp.int32, (8, 128), 0)
    cidx = jax.lax.broadcasted_iota(jnp.int32, (8, 128), 1)
    flat = ridx * 128 + cidx

    dot0 = _dot(tsn_s[pl.ds(0, 1), :], W_s[...]).reshape(8, 128)
    corr0 = jnp.zeros((1, 8), jnp.float32)

    def step(t, carry):
        s, r_prev, dot_stale, corr = carry
        nrm = nrm_s[...]
        h = h_s[...]
        gain = jnp.exp(_LAM * (1.0 - 1024.0 * h / s))
        beta_stale = dot_stale / nrm
        score_stale = gain * beta_stale
        c = jnp.max(corr)
        beta_patch = c / nrm
        score_patch = gain * beta_patch
        oh_prev = flat == r_prev
        score = jnp.where(oh_prev, score_patch, score_stale)
        n_star = jnp.argmax(score).astype(jnp.int32)
        onehot = flat == n_star
        beta_sel = jnp.where(oh_prev, beta_patch, beta_stale)
        bstar = jnp.sum(jnp.where(onehot, beta_sel, 0.0))
        hstar = jnp.sum(jnp.where(onehot, h, 0.0))
        alpha = 0.01 / (1.0 + hstar / 20000.0)
        tsn = tsn_s[pl.ds(t, 1), :]
        tnext = jnp.minimum(t + 1, _T - 1)
        tsn_next = tsn_s[pl.ds(tnext, 1), :]
        dot_next = _dot(tsn_next, W_s[...]).reshape(8, 128)
        Ck = W_s[pl.ds(n_star, 1), :]
        new_row = Ck + alpha * bstar * (tsn - Ck)
        W_s[pl.ds(n_star, 1), :] = new_row
        corr_next = _dot(tsn_next, jnp.broadcast_to(new_row, (8, _D)))
        nn = jnp.sqrt(jnp.sum(new_row * new_row))
        nrm_s[...] = jnp.where(onehot, nn, nrm)
        h_s[...] = jnp.where(onehot, h + 1.0, h)
        return (s + 1.0, n_star, dot_next, corr_next)

    s0 = jnp.sum(h_ref[...])
    carry = (s0, jnp.int32(-1), dot0, corr0)
    _, n_last, _, _ = jax.lax.fori_loop(0, _T, step, carry, unroll=2)
    out_ref[0, 0] = n_last


def kernel(all_ts, W, cumhisto):
    h2 = cumhisto.reshape(8, 128)
    out = pl.pallas_call(
        _body,
        out_shape=jax.ShapeDtypeStruct((1, 1), jnp.int32),
        in_specs=[
            pl.BlockSpec(memory_space=pltpu.VMEM),
            pl.BlockSpec(memory_space=pltpu.VMEM),
            pl.BlockSpec(memory_space=pltpu.VMEM),
        ],
        out_specs=pl.BlockSpec(memory_space=pltpu.SMEM),
        scratch_shapes=[
            pltpu.VMEM((_T, _D), jnp.float32),
            pltpu.VMEM((_N, _D), jnp.float32),
            pltpu.VMEM((8, 128), jnp.float32),
            pltpu.VMEM((8, 128), jnp.float32),
        ],
    )(all_ts, W, h2)
    return out[0, 0]
